# Initial kernel scaffold; baseline (speedup 1.0000x reference)
import jax
import jax.numpy as jnp
from jax.experimental import pallas as pl


def kernel(x, z_proxy, W, b):
    raise NotImplementedError("write your pallas kernel here")


# speedup vs baseline: 3.1589x; 3.1589x over previous
import functools

import jax
import jax.numpy as jnp
from jax import lax
from jax.experimental import pallas as pl
from jax.experimental.pallas import tpu as pltpu
from jax.experimental.pallas import tpu_sc as plsc

Z = 256
N = 100000
P = 8
C = 4
S = 50
K = 256
BLK = 4096
G = 25
NPAD = G * BLK
NEG = -1.0e30


def _att_loss_body(x_ref, zp_ref, w_ref, b_ref, att_ref, loss_ref):
    i = pl.program_id(0)
    xb = x_ref[0]
    zp = zp_ref[...]
    zp_mean = jnp.mean(zp, axis=1)
    ps_norm = jnp.sqrt(jnp.sum(zp_mean * zp_mean, axis=1, keepdims=True))
    ps = zp_mean / jnp.maximum(ps_norm, 1e-12)

    xn = jnp.sqrt(jnp.sum(xb * xb, axis=1, keepdims=True))
    zn = xb / jnp.maximum(xn, 1e-12)

    att = lax.dot_general(zn, ps, (((1,), (1,)), ((), ())),
                          preferred_element_type=jnp.float32)
    row = i * BLK + lax.broadcasted_iota(jnp.int32, (BLK, 1), 0)
    att = jnp.where(row < N, att, NEG)
    att_ref[...] = att

    @pl.when(i == 0)
    def _():
        logits = lax.dot_general(zp_mean, w_ref[...], (((1,), (0,)), ((), ())),
                                 preferred_element_type=jnp.float32)
        logits = logits + b_ref[...]
        haz = jax.nn.sigmoid(logits)
        q = 1.0 - haz
        s0 = q[:, 0:1]
        s1 = s0 * q[:, 1:2]
        s2 = s1 * q[:, 2:3]
        s3 = s2 * q[:, 3:4]
        ones = jnp.ones((P, 1), jnp.float32)
        s_pad = jnp.concatenate([ones, s0, s1, s2, s3], axis=1)
        r = lax.broadcasted_iota(jnp.int32, (P, 1), 0)
        y = r % C
        col5 = lax.broadcasted_iota(jnp.int32, (P, C + 1), 1)
        col4 = lax.broadcasted_iota(jnp.int32, (P, C), 1)
        eps = 1e-7
        s_prev = jnp.maximum(
            jnp.sum(jnp.where(col5 == y, s_pad, 0.0), axis=1, keepdims=True), eps)
        s_this = jnp.maximum(
            jnp.sum(jnp.where(col5 == y + 1, s_pad, 0.0), axis=1, keepdims=True), eps)
        h_this = jnp.maximum(
            jnp.sum(jnp.where(col4 == y, haz, 0.0), axis=1, keepdims=True), eps)
        cflag = jnp.where(r >= C, 1.0, 0.0)
        unc = -(1.0 - cflag) * (jnp.log(s_prev) + jnp.log(h_this))
        cen = -cflag * jnp.log(s_this)
        loss = jnp.sum(0.5 * (cen + unc) + 0.5 * unc, axis=0, keepdims=True)
        loss_ref[...] = loss


def _att_loss_call(x, z_proxy, W, b2):
    return pl.pallas_call(
        _att_loss_body,
        grid=(G,),
        in_specs=[
            pl.BlockSpec((1, BLK, Z), lambda i: (0, i, 0)),
            pl.BlockSpec((P, S, Z), lambda i: (0, 0, 0)),
            pl.BlockSpec((Z, C), lambda i: (0, 0)),
            pl.BlockSpec((1, C), lambda i: (0, 0)),
        ],
        out_specs=[
            pl.BlockSpec((BLK, P), lambda i: (i, 0)),
            pl.BlockSpec((1, 1), lambda i: (0, 0)),
        ],
        out_shape=[
            jax.ShapeDtypeStruct((NPAD, P), jnp.float32),
            jax.ShapeDtypeStruct((1, 1), jnp.float32),
        ],
    )(x, z_proxy, W, b2)


NR = (NPAD * P) // 1024


def _topk_body(att_ref, idx_ref, pos_ref, att_s):
    att = att_ref[...]
    att_s[...] = att
    m0 = jnp.max(att, axis=1, keepdims=True)

    rows_iota = lax.broadcasted_iota(jnp.int32, (NR, 1), 0)
    lane_iota = lax.broadcasted_iota(jnp.int32, (1, 1024), 1)
    k_iota = lax.broadcasted_iota(jnp.int32, (1, K), 1)

    def body(k, carry):
        m, flat_arr, val_arr = carry
        i_star = jnp.argmax(m)
        rowv = att_s[pl.ds(i_star, 1), :]
        j_star = jnp.argmax(rowv)
        val = jnp.max(rowv)
        new_row = jnp.where(lane_iota == j_star, NEG, rowv)
        att_s[pl.ds(i_star, 1), :] = new_row
        new_max = jnp.max(new_row)
        m = jnp.where(rows_iota == i_star, new_max, m)
        flat = i_star * 1024 + j_star
        flat_arr = jnp.where(k_iota == k, flat, flat_arr)
        val_arr = jnp.where(k_iota == k, val, val_arr)
        return m, flat_arr, val_arr

    _, flat_arr, val_arr = lax.fori_loop(
        0, K, body,
        (m0, jnp.zeros((1, K), jnp.int32), jnp.full((1, K), NEG, jnp.float32)))

    p_arr = flat_arr % P
    n_arr = flat_arr // P
    best_c = jnp.int32(0)
    best_cnt = jnp.sum(jnp.where(p_arr == 0, 1, 0))
    for c in range(1, P):
        cnt = jnp.sum(jnp.where(p_arr == c, 1, 0))
        better = cnt > best_cnt
        best_c = jnp.where(better, jnp.int32(c), best_c)
        best_cnt = jnp.maximum(cnt, best_cnt)
    pos_ref[...] = best_c * jnp.ones((1, 1), jnp.int32)

    flat2 = p_arr * N + n_arr

    def body2(k, carry):
        vrem, n_out = carry
        vmax = jnp.max(vrem)
        cand = vrem == vmax
        f2m = jnp.min(jnp.where(cand, flat2, jnp.int32(2147483647)))
        n_k = f2m % N
        n_out = jnp.where(k_iota == k, n_k, n_out)
        vrem = jnp.where(flat2 == f2m, -3.0e38, vrem)
        return vrem, n_out

    _, n_out = lax.fori_loop(0, K, body2,
                             (val_arr, jnp.zeros((1, K), jnp.int32)))
    idx_ref[...] = n_out


def _topk_call(att2):
    return pl.pallas_call(
        _topk_body,
        out_shape=[
            jax.ShapeDtypeStruct((1, K), jnp.int32),
            jax.ShapeDtypeStruct((1, 1), jnp.int32),
        ],
        scratch_shapes=[pltpu.VMEM((NR, 1024), jnp.float32)],
    )(att2)


NC = 2
NS = 16
NW = NC * NS
BPW = K // NW


@functools.cache
def _make_sc_gather():
    mesh = plsc.VectorSubcoreMesh(core_axis_name="c", subcore_axis_name="s")

    @functools.partial(
        pl.kernel, mesh=mesh,
        out_type=jax.ShapeDtypeStruct((K, Z), jnp.float32),
        scratch_types=[
            pltpu.VMEM((BPW,), jnp.int32),
            pltpu.VMEM((BPW, Z), jnp.float32),
            pltpu.SemaphoreType.DMA,
        ],
    )
    def gk(table_hbm, idx_hbm, out_hbm, idx_v, rows_v, sem):
        wid = lax.axis_index("s") * NC + lax.axis_index("c")
        base = wid * BPW
        pltpu.sync_copy(idx_hbm.at[pl.ds(base, BPW)], idx_v)
        pltpu.async_copy(table_hbm.at[idx_v], rows_v, sem).wait()
        pltpu.sync_copy(rows_v, out_hbm.at[pl.ds(base, BPW)])

    return gk


def _sc_gather(x2, idx):
    return _make_sc_gather()(x2, idx)


def kernel(x, z_proxy, W, b):
    att, loss = _att_loss_call(x, z_proxy, W, b.reshape(1, C))
    att2 = att.reshape(NR, 1024)
    idx2d, pos2d = _topk_call(att2)
    idx = idx2d.reshape(K)
    x2 = x.reshape(N, Z)
    z_topk = _sc_gather(x2, idx).reshape(1, K, Z)
    pos = pos2d[0, 0]
    proxy_pos = lax.dynamic_index_in_dim(z_proxy, pos, axis=0, keepdims=True)
    return loss.reshape(()), z_topk, proxy_pos


# speedup vs baseline: 3.4111x; 1.0798x over previous
import functools

import jax
import jax.numpy as jnp
from jax import lax
from jax.experimental import pallas as pl
from jax.experimental.pallas import tpu as pltpu
from jax.experimental.pallas import tpu_sc as plsc

Z = 256
N = 100000
P = 8
C = 4
S = 50
K = 256
BLK = 4096
G = 25
NPAD = G * BLK
NEG = -1.0e30


def _att_loss_body(x_ref, zp_ref, w_ref, b_ref, att_ref, loss_ref):
    i = pl.program_id(0)
    xb = x_ref[0]
    zp = zp_ref[...]
    zp_mean = jnp.mean(zp, axis=1)
    ps_norm = jnp.sqrt(jnp.sum(zp_mean * zp_mean, axis=1, keepdims=True))
    ps = zp_mean / jnp.maximum(ps_norm, 1e-12)

    xn = jnp.sqrt(jnp.sum(xb * xb, axis=1, keepdims=True))
    zn = xb / jnp.maximum(xn, 1e-12)

    att = lax.dot_general(zn, ps, (((1,), (1,)), ((), ())),
                          preferred_element_type=jnp.float32)
    row = i * BLK + lax.broadcasted_iota(jnp.int32, (BLK, 1), 0)
    att = jnp.where(row < N, att, NEG)
    att_ref[...] = att

    @pl.when(i == 0)
    def _():
        logits = lax.dot_general(zp_mean, w_ref[...], (((1,), (0,)), ((), ())),
                                 preferred_element_type=jnp.float32)
        logits = logits + b_ref[...]
        haz = jax.nn.sigmoid(logits)
        q = 1.0 - haz
        s0 = q[:, 0:1]
        s1 = s0 * q[:, 1:2]
        s2 = s1 * q[:, 2:3]
        s3 = s2 * q[:, 3:4]
        ones = jnp.ones((P, 1), jnp.float32)
        s_pad = jnp.concatenate([ones, s0, s1, s2, s3], axis=1)
        r = lax.broadcasted_iota(jnp.int32, (P, 1), 0)
        y = r % C
        col5 = lax.broadcasted_iota(jnp.int32, (P, C + 1), 1)
        col4 = lax.broadcasted_iota(jnp.int32, (P, C), 1)
        eps = 1e-7
        s_prev = jnp.maximum(
            jnp.sum(jnp.where(col5 == y, s_pad, 0.0), axis=1, keepdims=True), eps)
        s_this = jnp.maximum(
            jnp.sum(jnp.where(col5 == y + 1, s_pad, 0.0), axis=1, keepdims=True), eps)
        h_this = jnp.maximum(
            jnp.sum(jnp.where(col4 == y, haz, 0.0), axis=1, keepdims=True), eps)
        cflag = jnp.where(r >= C, 1.0, 0.0)
        unc = -(1.0 - cflag) * (jnp.log(s_prev) + jnp.log(h_this))
        cen = -cflag * jnp.log(s_this)
        loss = jnp.sum(0.5 * (cen + unc) + 0.5 * unc, axis=0, keepdims=True)
        loss_ref[...] = loss


def _att_loss_call(x, z_proxy, W, b2):
    return pl.pallas_call(
        _att_loss_body,
        grid=(G,),
        in_specs=[
            pl.BlockSpec((1, BLK, Z), lambda i: (0, i, 0)),
            pl.BlockSpec((P, S, Z), lambda i: (0, 0, 0)),
            pl.BlockSpec((Z, C), lambda i: (0, 0)),
            pl.BlockSpec((1, C), lambda i: (0, 0)),
        ],
        out_specs=[
            pl.BlockSpec((BLK, P), lambda i: (i, 0)),
            pl.BlockSpec((1, 1), lambda i: (0, 0)),
        ],
        out_shape=[
            jax.ShapeDtypeStruct((NPAD, P), jnp.float32),
            jax.ShapeDtypeStruct((1, 1), jnp.float32),
        ],
    )(x, z_proxy, W, b2)


NR = (NPAD * P) // 1024


def _topk_body(att_ref, idx_ref, pos_ref, att_s):
    att = att_ref[...]
    att_s[...] = att
    m0 = jnp.max(att, axis=1).reshape(1, NR)

    cols_iota = lax.broadcasted_iota(jnp.int32, (1, NR), 1)
    lane_iota = lax.broadcasted_iota(jnp.int32, (1, 1024), 1)
    k_iota = lax.broadcasted_iota(jnp.int32, (1, K), 1)

    def body(k, carry):
        m, flat_arr, val_arr = carry
        i_star = jnp.argmax(m)
        rowv = att_s[pl.ds(i_star, 1), :]
        j_star = jnp.argmax(rowv)
        val = jnp.max(rowv)
        new_row = jnp.where(lane_iota == j_star, NEG, rowv)
        att_s[pl.ds(i_star, 1), :] = new_row
        new_max = jnp.max(new_row)
        m = jnp.where(cols_iota == i_star, new_max, m)
        flat = i_star * 1024 + j_star
        flat_arr = jnp.where(k_iota == k, flat, flat_arr)
        val_arr = jnp.where(k_iota == k, val, val_arr)
        return m, flat_arr, val_arr

    _, flat_arr, val_arr = lax.fori_loop(
        0, K, body,
        (m0, jnp.zeros((1, K), jnp.int32), jnp.full((1, K), NEG, jnp.float32)))

    p_arr = flat_arr % P
    n_arr = flat_arr // P
    best_c = jnp.int32(0)
    best_cnt = jnp.sum(jnp.where(p_arr == 0, 1, 0))
    for c in range(1, P):
        cnt = jnp.sum(jnp.where(p_arr == c, 1, 0))
        better = cnt > best_cnt
        best_c = jnp.where(better, jnp.int32(c), best_c)
        best_cnt = jnp.maximum(cnt, best_cnt)
    pos_ref[...] = best_c * jnp.ones((1, 1), jnp.int32)

    flat2 = p_arr * N + n_arr

    def body2(k, carry):
        vrem, n_out = carry
        vmax = jnp.max(vrem)
        cand = vrem == vmax
        f2m = jnp.min(jnp.where(cand, flat2, jnp.int32(2147483647)))
        n_k = f2m % N
        n_out = jnp.where(k_iota == k, n_k, n_out)
        vrem = jnp.where(flat2 == f2m, -3.0e38, vrem)
        return vrem, n_out

    _, n_out = lax.fori_loop(0, K, body2,
                             (val_arr, jnp.zeros((1, K), jnp.int32)))
    idx_ref[...] = n_out


def _topk_call(att2):
    return pl.pallas_call(
        _topk_body,
        out_shape=[
            jax.ShapeDtypeStruct((1, K), jnp.int32),
            jax.ShapeDtypeStruct((1, 1), jnp.int32),
        ],
        scratch_shapes=[pltpu.VMEM((NR, 1024), jnp.float32)],
    )(att2)


NC = 2
NS = 16
NW = NC * NS
BPW = K // NW


@functools.cache
def _make_sc_gather():
    mesh = plsc.VectorSubcoreMesh(core_axis_name="c", subcore_axis_name="s")

    @functools.partial(
        pl.kernel, mesh=mesh,
        out_type=jax.ShapeDtypeStruct((K, Z), jnp.float32),
        scratch_types=[
            pltpu.VMEM((BPW,), jnp.int32),
            pltpu.VMEM((BPW, Z), jnp.float32),
            pltpu.SemaphoreType.DMA,
        ],
    )
    def gk(table_hbm, idx_hbm, out_hbm, idx_v, rows_v, sem):
        wid = lax.axis_index("s") * NC + lax.axis_index("c")
        base = wid * BPW
        pltpu.sync_copy(idx_hbm.at[pl.ds(base, BPW)], idx_v)
        pltpu.async_copy(table_hbm.at[idx_v], rows_v, sem).wait()
        pltpu.sync_copy(rows_v, out_hbm.at[pl.ds(base, BPW)])

    return gk


def _sc_gather(x2, idx):
    return _make_sc_gather()(x2, idx)


def kernel(x, z_proxy, W, b):
    att, loss = _att_loss_call(x, z_proxy, W, b.reshape(1, C))
    att2 = att.reshape(NR, 1024)
    idx2d, pos2d = _topk_call(att2)
    idx = idx2d.reshape(K)
    x2 = x.reshape(N, Z)
    z_topk = _sc_gather(x2, idx).reshape(1, K, Z)
    pos = pos2d[0, 0]
    proxy_pos = lax.dynamic_index_in_dim(z_proxy, pos, axis=0, keepdims=True)
    return loss.reshape(()), z_topk, proxy_pos


# speedup vs baseline: 3.5000x; 1.0261x over previous
import functools

import jax
import jax.numpy as jnp
from jax import lax
from jax.experimental import pallas as pl
from jax.experimental.pallas import tpu as pltpu
from jax.experimental.pallas import tpu_sc as plsc

Z = 256
N = 100000
P = 8
C = 4
S = 50
K = 256
BLK = 4096
G = 25
NPAD = G * BLK
NEG = -1.0e30


def _att_loss_body(x_ref, zp_ref, w_ref, b_ref, att_ref, loss_ref):
    i = pl.program_id(0)
    xb = x_ref[0]
    zp = zp_ref[...]
    zp_mean = jnp.mean(zp, axis=1)
    ps_norm = jnp.sqrt(jnp.sum(zp_mean * zp_mean, axis=1, keepdims=True))
    ps = zp_mean / jnp.maximum(ps_norm, 1e-12)

    xn = jnp.sqrt(jnp.sum(xb * xb, axis=1, keepdims=True))
    zn = xb / jnp.maximum(xn, 1e-12)

    att = lax.dot_general(zn, ps, (((1,), (1,)), ((), ())),
                          preferred_element_type=jnp.float32)
    row = i * BLK + lax.broadcasted_iota(jnp.int32, (BLK, 1), 0)
    att = jnp.where(row < N, att, NEG)
    att_ref[...] = att

    @pl.when(i == 0)
    def _():
        logits = lax.dot_general(zp_mean, w_ref[...], (((1,), (0,)), ((), ())),
                                 preferred_element_type=jnp.float32)
        logits = logits + b_ref[...]
        haz = jax.nn.sigmoid(logits)
        q = 1.0 - haz
        s0 = q[:, 0:1]
        s1 = s0 * q[:, 1:2]
        s2 = s1 * q[:, 2:3]
        s3 = s2 * q[:, 3:4]
        ones = jnp.ones((P, 1), jnp.float32)
        s_pad = jnp.concatenate([ones, s0, s1, s2, s3], axis=1)
        r = lax.broadcasted_iota(jnp.int32, (P, 1), 0)
        y = r % C
        col5 = lax.broadcasted_iota(jnp.int32, (P, C + 1), 1)
        col4 = lax.broadcasted_iota(jnp.int32, (P, C), 1)
        eps = 1e-7
        s_prev = jnp.maximum(
            jnp.sum(jnp.where(col5 == y, s_pad, 0.0), axis=1, keepdims=True), eps)
        s_this = jnp.maximum(
            jnp.sum(jnp.where(col5 == y + 1, s_pad, 0.0), axis=1, keepdims=True), eps)
        h_this = jnp.maximum(
            jnp.sum(jnp.where(col4 == y, haz, 0.0), axis=1, keepdims=True), eps)
        cflag = jnp.where(r >= C, 1.0, 0.0)
        unc = -(1.0 - cflag) * (jnp.log(s_prev) + jnp.log(h_this))
        cen = -cflag * jnp.log(s_this)
        loss = jnp.sum(0.5 * (cen + unc) + 0.5 * unc, axis=0, keepdims=True)
        loss_ref[...] = loss


def _att_loss_call(x, z_proxy, W, b2):
    return pl.pallas_call(
        _att_loss_body,
        grid=(G,),
        in_specs=[
            pl.BlockSpec((1, BLK, Z), lambda i: (0, i, 0)),
            pl.BlockSpec((P, S, Z), lambda i: (0, 0, 0)),
            pl.BlockSpec((Z, C), lambda i: (0, 0)),
            pl.BlockSpec((1, C), lambda i: (0, 0)),
        ],
        out_specs=[
            pl.BlockSpec((BLK, P), lambda i: (i, 0)),
            pl.BlockSpec((1, 1), lambda i: (0, 0)),
        ],
        out_shape=[
            jax.ShapeDtypeStruct((NPAD, P), jnp.float32),
            jax.ShapeDtypeStruct((1, 1), jnp.float32),
        ],
    )(x, z_proxy, W, b2)


NR = (NPAD * P) // 1024


def _topk_body(att_ref, idx_ref, pos_ref, att_s):
    att = att_ref[...]
    att_s[...] = att
    m0 = jnp.max(att, axis=1).reshape(1, NR)

    cols_iota = lax.broadcasted_iota(jnp.int32, (1, NR), 1)
    lane_iota = lax.broadcasted_iota(jnp.int32, (1, 1024), 1)
    k_iota = lax.broadcasted_iota(jnp.int32, (1, K), 1)

    big = jnp.int32(2147483647)

    def body(k, carry):
        m, flat_arr, val_arr = carry
        vmax = jnp.max(m, axis=1, keepdims=True)
        i_star = jnp.min(jnp.where(m == vmax, cols_iota, big))
        rowv = att_s[pl.ds(i_star, 1), :]
        j_star = jnp.min(jnp.where(rowv == vmax, lane_iota, big),
                         axis=1, keepdims=True)
        new_row = jnp.where(lane_iota == j_star, NEG, rowv)
        att_s[pl.ds(i_star, 1), :] = new_row
        new_max = jnp.max(new_row, axis=1, keepdims=True)
        m = jnp.where(cols_iota == i_star, new_max, m)
        flat = i_star * 1024 + j_star
        flat_arr = jnp.where(k_iota == k, flat, flat_arr)
        val_arr = jnp.where(k_iota == k, vmax, val_arr)
        return m, flat_arr, val_arr

    _, flat_arr, val_arr = lax.fori_loop(
        0, K, body,
        (m0, jnp.zeros((1, K), jnp.int32), jnp.full((1, K), NEG, jnp.float32)))

    p_arr = flat_arr % P
    n_arr = flat_arr // P
    best_c = jnp.int32(0)
    best_cnt = jnp.sum(jnp.where(p_arr == 0, 1, 0))
    for c in range(1, P):
        cnt = jnp.sum(jnp.where(p_arr == c, 1, 0))
        better = cnt > best_cnt
        best_c = jnp.where(better, jnp.int32(c), best_c)
        best_cnt = jnp.maximum(cnt, best_cnt)
    pos_ref[...] = best_c * jnp.ones((1, 1), jnp.int32)

    flat2 = p_arr * N + n_arr

    def body2(k, carry):
        vrem, n_out = carry
        vmax = jnp.max(vrem, axis=1, keepdims=True)
        f2m = jnp.min(jnp.where(vrem == vmax, flat2, big),
                      axis=1, keepdims=True)
        n_k = f2m % N
        n_out = jnp.where(k_iota == k, n_k, n_out)
        vrem = jnp.where(flat2 == f2m, -3.0e38, vrem)
        return vrem, n_out

    _, n_out = lax.fori_loop(0, K, body2,
                             (val_arr, jnp.zeros((1, K), jnp.int32)))
    idx_ref[...] = n_out


def _topk_call(att2):
    return pl.pallas_call(
        _topk_body,
        out_shape=[
            jax.ShapeDtypeStruct((1, K), jnp.int32),
            jax.ShapeDtypeStruct((1, 1), jnp.int32),
        ],
        scratch_shapes=[pltpu.VMEM((NR, 1024), jnp.float32)],
    )(att2)


NC = 2
NS = 16
NW = NC * NS
BPW = K // NW


@functools.cache
def _make_sc_gather():
    mesh = plsc.VectorSubcoreMesh(core_axis_name="c", subcore_axis_name="s")

    @functools.partial(
        pl.kernel, mesh=mesh,
        out_type=jax.ShapeDtypeStruct((K, Z), jnp.float32),
        scratch_types=[
            pltpu.VMEM((BPW,), jnp.int32),
            pltpu.VMEM((BPW, Z), jnp.float32),
            pltpu.SemaphoreType.DMA,
        ],
    )
    def gk(table_hbm, idx_hbm, out_hbm, idx_v, rows_v, sem):
        wid = lax.axis_index("s") * NC + lax.axis_index("c")
        base = wid * BPW
        pltpu.sync_copy(idx_hbm.at[pl.ds(base, BPW)], idx_v)
        pltpu.async_copy(table_hbm.at[idx_v], rows_v, sem).wait()
        pltpu.sync_copy(rows_v, out_hbm.at[pl.ds(base, BPW)])

    return gk


def _sc_gather(x2, idx):
    return _make_sc_gather()(x2, idx)


def kernel(x, z_proxy, W, b):
    att, loss = _att_loss_call(x, z_proxy, W, b.reshape(1, C))
    att2 = att.reshape(NR, 1024)
    idx2d, pos2d = _topk_call(att2)
    idx = idx2d.reshape(K)
    x2 = x.reshape(N, Z)
    z_topk = _sc_gather(x2, idx).reshape(1, K, Z)
    pos = pos2d[0, 0]
    proxy_pos = lax.dynamic_index_in_dim(z_proxy, pos, axis=0, keepdims=True)
    return loss.reshape(()), z_topk, proxy_pos
